# Initial kernel scaffold; baseline (speedup 1.0000x reference)
import functools

import jax
import jax.numpy as jnp
from jax import lax
from jax.experimental import pallas as pl
from jax.experimental.pallas import tpu as pltpu
from jax.experimental.pallas import tpu_sc as plsc

NC, NS = 2, 16
NW = NC * NS
SCK = 128
LAT = 128


def _rup(x, m):
    return (x + m - 1) // m * m


def _sc_pass(Epad, n_idx, gathers, scatters):
    per_w = Epad // NW
    n_ch = per_w // SCK
    n_g, n_s = len(gathers), len(scatters)
    used = sorted({s for _, s in gathers} | {s[2] for s in scatters})
    spos = {s: k for k, s in enumerate(used)}
    in_ts = [t for t in range(n_s) if scatters[t][3] == 'in']
    vpos = {t: u for u, t in enumerate(in_ts)}
    need_ones = any(s[3] == 'o' for s in scatters)
    mesh = plsc.VectorSubcoreMesh(core_axis_name="c", subcore_axis_name="s",
                                  num_cores=NC, num_subcores=NS)

    def run(idx_arrays, tables, in_values):
        idx_r = [a.reshape(NW, n_ch, SCK) for a in idx_arrays]
        zeros_l = [jnp.zeros((s[0] // NS, s[1]), jnp.float32) for s in scatters]
        ones_l = [jnp.ones((SCK, 16), jnp.float32)] if need_ones else []
        out_type = tuple(
            [jax.ShapeDtypeStruct((Epad, D), jnp.float32) for D, _ in gathers]
            + [jax.ShapeDtypeStruct((NC, s[0], s[1]), jnp.float32)
               for s in scatters])
        scratch = ([pltpu.VMEM((n_ch, SCK), jnp.int32) for _ in used]
                   + [pltpu.VMEM((SCK, D), jnp.float32) for D, _ in gathers]
                   + [pltpu.VMEM((SCK, scatters[t][1]), jnp.float32)
                      for t in in_ts]
                   + ([pltpu.VMEM((SCK, 16), jnp.float32)] if need_ones else [])
                   + [pltpu.VMEM_SHARED((s[0], s[1]), jnp.float32)
                      for s in scatters]
                   + [pltpu.SemaphoreType.DMA])

        def body(*refs):
            p = 0
            idx_in = refs[p:p + n_idx]; p += n_idx
            tbl_in = refs[p:p + n_g]; p += n_g
            val_in = refs[p:p + len(in_ts)]; p += len(in_ts)
            ones_in = refs[p:p + len(ones_l)]; p += len(ones_l)
            zer_in = refs[p:p + n_s]; p += n_s
            gout = refs[p:p + n_g]; p += n_g
            sacc = refs[p:p + n_s]; p += n_s
            idx_b = refs[p:p + len(used)]; p += len(used)
            gbuf = refs[p:p + n_g]; p += n_g
            vbuf = refs[p:p + len(in_ts)]; p += len(in_ts)
            ones_b = None
            if need_ones:
                ones_b = refs[p]; p += 1
            smem = refs[p:p + n_s]; p += n_s
            sem = refs[p]

            cid = lax.axis_index("c")
            sid = lax.axis_index("s")
            wid = sid * NC + cid
            for t in range(n_s):
                nt = scatters[t][0] // NS
                pltpu.sync_copy(zer_in[t], smem[t].at[pl.ds(sid * nt, nt)])
            if need_ones:
                pltpu.sync_copy(ones_in[0], ones_b)
            for k in range(len(used)):
                pltpu.sync_copy(idx_in[used[k]].at[wid], idx_b[k])
            plsc.subcore_barrier()
            base = wid * per_w

            def chunk(c, carry):
                off = pl.multiple_of(base + c * SCK, SCK)
                for k, (D, slot) in enumerate(gathers):
                    pltpu.async_copy(tbl_in[k].at[idx_b[spos[slot]].at[c]],
                                     gbuf[k], sem).wait()
                    pltpu.sync_copy(gbuf[k], gout[k].at[pl.ds(off, SCK)])
                for t, (Npt, D, slot, src) in enumerate(scatters):
                    if src == 'in':
                        pltpu.sync_copy(val_in[vpos[t]].at[pl.ds(off, SCK)],
                                        vbuf[vpos[t]])
                        sb = vbuf[vpos[t]]
                    elif src == 'o':
                        sb = ones_b
                    else:
                        sb = gbuf[src[1]]
                    pltpu.sync_copy(sb, smem[t].at[idx_b[spos[slot]].at[c]],
                                    add=True)
                return carry

            lax.fori_loop(0, n_ch, chunk, 0)
            plsc.subcore_barrier()
            for t in range(n_s):
                nt = scatters[t][0] // NS
                pltpu.sync_copy(smem[t].at[pl.ds(sid * nt, nt)],
                                sacc[t].at[cid, pl.ds(sid * nt, nt)])

        fn = pl.kernel(body, out_type=out_type, mesh=mesh,
                       scratch_types=scratch)
        return fn(*(idx_r + list(tables) + list(in_values) + ones_l + zeros_l))

    return run


def _full(a):
    nd = a.ndim
    return pl.BlockSpec(a.shape, lambda r: (0,) * nd)


def _row(blk, d):
    return pl.BlockSpec((blk, d), lambda r: (r, 0))


def _mlp_tail(x, w2, b2, w3, b3, w4, b4, g, be):
    x = jnp.maximum(jnp.dot(x, w2, preferred_element_type=jnp.float32) + b2, 0.)
    x = jnp.maximum(jnp.dot(x, w3, preferred_element_type=jnp.float32) + b3, 0.)
    x = jnp.dot(x, w4, preferred_element_type=jnp.float32) + b4
    mu = jnp.mean(x, axis=-1, keepdims=True)
    var = jnp.mean((x - mu) ** 2, axis=-1, keepdims=True)
    return (x - mu) * lax.rsqrt(var + 1e-5) * g + be


def _matmul2(h, wa, wb, blk=512):
    n = h.shape[0]

    def body(h_, wa_, wb_, a_, b_):
        x = h_[...]
        a_[...] = jnp.dot(x, wa_[...], preferred_element_type=jnp.float32)
        b_[...] = jnp.dot(x, wb_[...], preferred_element_type=jnp.float32)

    return pl.pallas_call(
        body, grid=(pl.cdiv(n, blk),),
        in_specs=[_row(blk, LAT), _full(wa), _full(wb)],
        out_specs=[_row(blk, LAT), _row(blk, LAT)],
        out_shape=[jax.ShapeDtypeStruct((n, LAT), jnp.float32)] * 2,
    )(h, wa, wb)


def _edge_mlp(Zi, Zj, Pi, Pj, W, blk=512):
    ep = Zi.shape[0]
    wl = [W['wg'], W['b1'], W['w2'], W['b2'], W['w3'], W['b3'], W['w4'],
          W['b4'], W['g'], W['be']]

    def body(zi, zj, pi, pj, wg, b1, w2, b2, w3, b3, w4, b4, g, be, out):
        dif = pi[...] - pj[...]
        nrm = jnp.sqrt(jnp.sum(dif * dif, axis=-1, keepdims=True))
        lane = lax.broadcasted_iota(jnp.int32, (blk, 16), 1)
        x16 = jnp.where(lane == 3, nrm, dif)
        x = (zi[...] + zj[...] + b1[...]
             + jnp.dot(x16, wg[...], preferred_element_type=jnp.float32))
        x = jnp.maximum(x, 0.)
        out[...] = _mlp_tail(x, w2[...], b2[...], w3[...], b3[...], w4[...],
                             b4[...], g[...], be[...])

    return pl.pallas_call(
        body, grid=(ep // blk,),
        in_specs=[_row(blk, LAT), _row(blk, LAT), _row(blk, 16),
                  _row(blk, 16)] + [_full(a) for a in wl],
        out_specs=_row(blk, LAT),
        out_shape=jax.ShapeDtypeStruct((ep, LAT), jnp.float32),
    )(Zi, Zj, Pi, Pj, *wl)


def _node_mlp(h, a0, a1, W, skip=None, blk=512):
    n = h.shape[0]
    wl = [W['nwh'], W['nwag'], W['nb1'], W['nw2'], W['nb2'], W['nw3'],
          W['nb3'], W['nw4'], W['nb4'], W['ng'], W['nbe']]
    has_skip = skip is not None

    def body(*refs):
        h_, a0_, a1_ = refs[0], refs[1], refs[2]
        k = 3
        sk = None
        if has_skip:
            sk = refs[3]
            k = 4
        wh, wag, b1, w2, b2, w3, b3, w4, b4, g, be = refs[k:k + 11]
        out = refs[k + 11]
        hv = h_[...]
        x = (jnp.dot(hv, wh[...], preferred_element_type=jnp.float32)
             + jnp.dot(a0_[...] + a1_[...], wag[...],
                       preferred_element_type=jnp.float32) + b1[...])
        x = jnp.maximum(x, 0.)
        res = _mlp_tail(x, w2[...], b2[...], w3[...], b3[...], w4[...],
                        b4[...], g[...], be[...])
        y = hv + res
        if has_skip:
            y = y + sk[...]
        out[...] = y

    arrs = [h, a0, a1] + ([skip] if has_skip else [])
    return pl.pallas_call(
        body, grid=(pl.cdiv(n, blk),),
        in_specs=[_row(blk, LAT)] * len(arrs) + [_full(a) for a in wl],
        out_specs=_row(blk, LAT),
        out_shape=jax.ShapeDtypeStruct((n, LAT), jnp.float32),
    )(*(arrs + wl))


def _rowwise(fn, out_dims, arrs, blk=256):
    n = arrs[0].shape[0]

    def body(*refs):
        ins, outs = refs[:len(arrs)], refs[len(arrs):]
        vals = fn(*[x[...] for x in ins])
        if not isinstance(vals, tuple):
            vals = (vals,)
        for o, v in zip(outs, vals):
            o[...] = v

    res = pl.pallas_call(
        body, grid=(pl.cdiv(n, blk),),
        in_specs=[_row(blk, a.shape[1]) for a in arrs],
        out_specs=[_row(blk, d) for d in out_dims],
        out_shape=[jax.ShapeDtypeStruct((n, d), jnp.float32)
                   for d in out_dims],
    )(*arrs)
    return res if len(out_dims) > 1 else res[0]


def _add2(a, b):
    return _rowwise(lambda x, y: x + y, [a.shape[1]], [a, b])


def _prep_gmp(p):
    ew, eb = p['edge']['W'], p['edge']['b']
    nw, nb = p['node']['W'], p['node']['b']
    r = lambda v: v.reshape(1, -1)
    w1 = ew[0]
    geo = w1[2 * LAT:]
    return {
        'wa': w1[:LAT], 'wb': w1[LAT:2 * LAT],
        'wg': jnp.pad(geo, ((0, 16 - geo.shape[0]), (0, 0))),
        'b1': r(eb[0]), 'w2': ew[1], 'b2': r(eb[1]), 'w3': ew[2],
        'b3': r(eb[2]), 'w4': ew[3], 'b4': r(eb[3]),
        'g': r(p['edge']['g']), 'be': r(p['edge']['be']),
        'nwh': nw[0][:LAT], 'nwag': nw[0][LAT:], 'nb1': r(nb[0]),
        'nw2': nw[1], 'nb2': r(nb[1]), 'nw3': nw[2], 'nb3': r(nb[2]),
        'nw4': nw[3], 'nb4': r(nb[3]),
        'ng': r(p['node']['g']), 'nbe': r(p['node']['be']),
    }


def kernel(h, pos, m_ids_0, m_ids_1, m_gs_0, m_gs_1, m_gs_2, params):
    f32 = jnp.float32
    h = h.astype(f32)
    ns = (h.shape[0], m_ids_0.shape[0], m_ids_1.shape[0])
    gs = (m_gs_0, m_gs_1, m_gs_2)

    lv = []
    for l in range(3):
        e = gs[l].shape[1]
        n = ns[l]
        epad = _rup(e, NW * SCK)
        npd = _rup(n, 128)
        i, j = gs[l][0], gs[l][1]
        p0 = lambda a: jnp.pad(a, (0, epad - e))
        ps = lambda a: jnp.pad(a, (0, epad - e), constant_values=n)
        lv.append(dict(E=e, Epad=epad, N=n, Np=npd, ig=p0(i), jg=p0(j),
                       i_s=ps(i), js=ps(j)))

    pos16 = jnp.pad(pos.astype(f32), ((0, 0), (0, 16 - pos.shape[1])))
    dn = [_prep_gmp(params['down'][k]) for k in range(2)]
    upp = [_prep_gmp(params['up'][k]) for k in range(2)]
    bt = _prep_gmp(params['bottom'])

    def gmp(W, hh, p16, L, skip=None, with_deg=False):
        A, B = _matmul2(hh, W['wa'], W['wb'])
        g4 = _sc_pass(L['Epad'], 2,
                      gathers=((LAT, 0), (LAT, 1), (16, 0), (16, 1)),
                      scatters=())
        Zi, Zj, Pi, Pj = g4([L['ig'], L['jg']], [A, B, p16, p16], [])
        e = _edge_mlp(Zi, Zj, Pi, Pj, W)
        if with_deg:
            s = _sc_pass(L['Epad'], 2, gathers=(),
                         scatters=((L['Np'], LAT, 0, 'in'),
                                   (L['Np'], 16, 1, 'o')))
            aggp, degp = s([L['js'], L['i_s']], [], [e])
        else:
            s = _sc_pass(L['Epad'], 1, gathers=(),
                         scatters=((L['Np'], LAT, 0, 'in'),))
            (aggp,) = s([L['js']], [], [e])
            degp = None
        n = L['N']
        hout = _node_mlp(hh, aggp[0][:n], aggp[1][:n], W, skip)
        return hout, degp

    w16 = jnp.ones((ns[0], 16), f32)
    down_h, down_p, ecs = [], [], []
    hh, p16 = h, pos16
    for l in range(2):
        L = lv[l]
        n, n1 = L['N'], lv[l + 1]['N']
        h1, degp = gmp(dn[l], hh, p16, L, with_deg=True)
        down_h.append(h1)
        down_p.append(p16)
        nw16 = _rowwise(lambda w, d0, d1: w / jnp.maximum(d0 + d1, 1.0), [16],
                        [w16, degp[0][:n], degp[1][:n]])
        gsc = _sc_pass(L['Epad'], 2, gathers=((16, 0),),
                       scatters=((L['Np'], 16, 1, ('g', 0)),))
        ws, awp = gsc([L['ig'], L['js']], [nw16], [])
        aw = _rowwise(lambda a, b: a + b + 1e-12, [16],
                      [awp[0][:n], awp[1][:n]])
        g3 = _sc_pass(L['Epad'], 2,
                      gathers=((LAT, 0), (16, 0), (16, 1)), scatters=())
        he, pe, awj = g3([L['ig'], L['jg']], [h1, p16, aw], [])
        ec = _rowwise(lambda a, b: a / b, [16], [ws, awj])
        yh, yp = _rowwise(lambda x, q, c: (x * c[:, :1], q * c), [LAT, 16],
                          [he, pe, ec])
        s2 = _sc_pass(L['Epad'], 1, gathers=(),
                      scatters=((L['Np'], LAT, 0, 'in'),
                                (L['Np'], 16, 0, 'in')))
        php, ppp = s2([L['js']], [], [yh, yp])
        hh = _add2(php[0][:n1], php[1][:n1])
        p16 = _add2(ppp[0][:n1], ppp[1][:n1])
        w16 = aw[:n1]
        ecs.append(ec)

    hh, _ = gmp(bt, hh, p16, lv[2])

    for k in range(2):
        d = 1 - k
        L = lv[d]
        n = L['N']
        hfull = jnp.pad(hh, ((0, n - hh.shape[0]), (0, 0)))
        g1 = _sc_pass(L['Epad'], 1, gathers=((LAT, 0),), scatters=())
        (hjg,) = g1([L['jg']], [hfull], [])
        yu = _rowwise(lambda x, c: x * c[:, :1], [LAT], [hjg, ecs[d]])
        s1 = _sc_pass(L['Epad'], 1, gathers=(),
                      scatters=((L['Np'], LAT, 0, 'in'),))
        (unp,) = s1([L['i_s']], [], [yu])
        hin = _add2(unp[0][:n], unp[1][:n])
        hh, _ = gmp(upp[k], hin, down_p[d], L, skip=down_h[d])

    return hh


# speedup vs baseline: 1.2348x; 1.2348x over previous
import functools

import jax
import jax.numpy as jnp
from jax import lax
from jax.experimental import pallas as pl
from jax.experimental.pallas import tpu as pltpu
from jax.experimental.pallas import tpu_sc as plsc

NC, NS = 2, 16
NW = NC * NS
SCK = 128
LAT = 128


def _rup(x, m):
    return (x + m - 1) // m * m


def _sc_pass(Epad, n_idx, gathers, scatters):
    per_w = Epad // NW
    n_ch = per_w // SCK
    n_g, n_s = len(gathers), len(scatters)
    used = sorted({s for _, s in gathers} | {s[2] for s in scatters})
    spos = {s: k for k, s in enumerate(used)}
    in_ts = [t for t in range(n_s) if scatters[t][3] == 'in']
    vpos = {t: u for u, t in enumerate(in_ts)}
    need_ones = any(s[3] == 'o' for s in scatters)
    mesh = plsc.VectorSubcoreMesh(core_axis_name="c", subcore_axis_name="s",
                                  num_cores=NC, num_subcores=NS)

    def run(idx_arrays, tables, in_values):
        idx_r = [a.reshape(NW, n_ch, SCK) for a in idx_arrays]
        zeros_l = [jnp.zeros((s[0] // NS, s[1]), jnp.float32) for s in scatters]
        ones_l = [jnp.ones((SCK, LAT), jnp.float32)] if need_ones else []
        out_type = tuple(
            [jax.ShapeDtypeStruct((Epad, D), jnp.float32) for D, _ in gathers]
            + [jax.ShapeDtypeStruct((NC, s[0], s[1]), jnp.float32)
               for s in scatters])
        scratch = ([pltpu.VMEM((n_ch, SCK), jnp.int32) for _ in used]
                   + [pltpu.VMEM((SCK, D), jnp.float32) for D, _ in gathers]
                   + [pltpu.VMEM((SCK, scatters[t][1]), jnp.float32)
                      for t in in_ts]
                   + ([pltpu.VMEM((SCK, LAT), jnp.float32)] if need_ones else [])
                   + [pltpu.VMEM_SHARED((s[0], s[1]), jnp.float32)
                      for s in scatters]
                   + [pltpu.SemaphoreType.DMA])

        def body(*refs):
            p = 0
            idx_in = refs[p:p + n_idx]; p += n_idx
            tbl_in = refs[p:p + n_g]; p += n_g
            val_in = refs[p:p + len(in_ts)]; p += len(in_ts)
            ones_in = refs[p:p + len(ones_l)]; p += len(ones_l)
            zer_in = refs[p:p + n_s]; p += n_s
            gout = refs[p:p + n_g]; p += n_g
            sacc = refs[p:p + n_s]; p += n_s
            idx_b = refs[p:p + len(used)]; p += len(used)
            gbuf = refs[p:p + n_g]; p += n_g
            vbuf = refs[p:p + len(in_ts)]; p += len(in_ts)
            ones_b = None
            if need_ones:
                ones_b = refs[p]; p += 1
            smem = refs[p:p + n_s]; p += n_s
            sem = refs[p]

            cid = lax.axis_index("c")
            sid = lax.axis_index("s")
            wid = sid * NC + cid
            for t in range(n_s):
                nt = scatters[t][0] // NS
                pltpu.sync_copy(zer_in[t], smem[t].at[pl.ds(sid * nt, nt)])
            if need_ones:
                pltpu.sync_copy(ones_in[0], ones_b)
            for k in range(len(used)):
                pltpu.sync_copy(idx_in[used[k]].at[wid], idx_b[k])
            plsc.subcore_barrier()
            base = wid * per_w

            def chunk(c, carry):
                off = pl.multiple_of(base + c * SCK, SCK)
                for k, (D, slot) in enumerate(gathers):
                    pltpu.async_copy(tbl_in[k].at[idx_b[spos[slot]].at[c]],
                                     gbuf[k], sem).wait()
                    pltpu.sync_copy(gbuf[k], gout[k].at[pl.ds(off, SCK)])
                for t, (Npt, D, slot, src) in enumerate(scatters):
                    if src == 'in':
                        pltpu.sync_copy(val_in[vpos[t]].at[pl.ds(off, SCK)],
                                        vbuf[vpos[t]])
                        sb = vbuf[vpos[t]]
                    elif src == 'o':
                        sb = ones_b
                    else:
                        sb = gbuf[src[1]]
                    pltpu.sync_copy(sb, smem[t].at[idx_b[spos[slot]].at[c]],
                                    add=True)
                return carry

            lax.fori_loop(0, n_ch, chunk, 0)
            plsc.subcore_barrier()
            for t in range(n_s):
                nt = scatters[t][0] // NS
                pltpu.sync_copy(smem[t].at[pl.ds(sid * nt, nt)],
                                sacc[t].at[cid, pl.ds(sid * nt, nt)])

        fn = pl.kernel(body, out_type=out_type, mesh=mesh,
                       scratch_types=scratch)
        return fn(*(idx_r + list(tables) + list(in_values) + ones_l + zeros_l))

    return run


def _full(a):
    nd = a.ndim
    return pl.BlockSpec(a.shape, lambda r: (0,) * nd)


def _row(blk, d):
    return pl.BlockSpec((blk, d), lambda r: (r, 0))


def _mlp_tail(x, w2, b2, w3, b3, w4, b4, g, be):
    x = jnp.maximum(jnp.dot(x, w2, preferred_element_type=jnp.float32) + b2, 0.)
    x = jnp.maximum(jnp.dot(x, w3, preferred_element_type=jnp.float32) + b3, 0.)
    x = jnp.dot(x, w4, preferred_element_type=jnp.float32) + b4
    mu = jnp.mean(x, axis=-1, keepdims=True)
    var = jnp.mean((x - mu) ** 2, axis=-1, keepdims=True)
    return (x - mu) * lax.rsqrt(var + 1e-5) * g + be


def _matmul2(h, wa, wb, blk=512):
    n = h.shape[0]

    def body(h_, wa_, wb_, a_, b_):
        x = h_[...]
        a_[...] = jnp.dot(x, wa_[...], preferred_element_type=jnp.float32)
        b_[...] = jnp.dot(x, wb_[...], preferred_element_type=jnp.float32)

    return pl.pallas_call(
        body, grid=(pl.cdiv(n, blk),),
        in_specs=[_row(blk, LAT), _full(wa), _full(wb)],
        out_specs=[_row(blk, LAT), _row(blk, LAT)],
        out_shape=[jax.ShapeDtypeStruct((n, LAT), jnp.float32)] * 2,
    )(h, wa, wb)


def _edge_mlp(Zi, Zj, Pi, Pj, W, blk=512):
    ep = Zi.shape[0]
    wl = [W['wg'], W['b1'], W['w2'], W['b2'], W['w3'], W['b3'], W['w4'],
          W['b4'], W['g'], W['be']]

    def body(zi, zj, pi, pj, wg, b1, w2, b2, w3, b3, w4, b4, g, be, out):
        dif = pi[...] - pj[...]
        nrm = jnp.sqrt(jnp.sum(dif * dif, axis=-1, keepdims=True))
        lane = lax.broadcasted_iota(jnp.int32, (blk, LAT), 1)
        x16 = jnp.where(lane == 3, nrm, dif)
        x = (zi[...] + zj[...] + b1[...]
             + jnp.dot(x16, wg[...], preferred_element_type=jnp.float32))
        x = jnp.maximum(x, 0.)
        out[...] = _mlp_tail(x, w2[...], b2[...], w3[...], b3[...], w4[...],
                             b4[...], g[...], be[...])

    return pl.pallas_call(
        body, grid=(ep // blk,),
        in_specs=[_row(blk, LAT)] * 4 + [_full(a) for a in wl],
        out_specs=_row(blk, LAT),
        out_shape=jax.ShapeDtypeStruct((ep, LAT), jnp.float32),
    )(Zi, Zj, Pi, Pj, *wl)


def _node_mlp(h, a0, a1, W, skip=None, blk=512):
    n = h.shape[0]
    wl = [W['nwh'], W['nwag'], W['nb1'], W['nw2'], W['nb2'], W['nw3'],
          W['nb3'], W['nw4'], W['nb4'], W['ng'], W['nbe']]
    has_skip = skip is not None

    def body(*refs):
        h_, a0_, a1_ = refs[0], refs[1], refs[2]
        k = 3
        sk = None
        if has_skip:
            sk = refs[3]
            k = 4
        wh, wag, b1, w2, b2, w3, b3, w4, b4, g, be = refs[k:k + 11]
        out = refs[k + 11]
        hv = h_[...]
        x = (jnp.dot(hv, wh[...], preferred_element_type=jnp.float32)
             + jnp.dot(a0_[...] + a1_[...], wag[...],
                       preferred_element_type=jnp.float32) + b1[...])
        x = jnp.maximum(x, 0.)
        res = _mlp_tail(x, w2[...], b2[...], w3[...], b3[...], w4[...],
                        b4[...], g[...], be[...])
        y = hv + res
        if has_skip:
            y = y + sk[...]
        out[...] = y

    arrs = [h, a0, a1] + ([skip] if has_skip else [])
    return pl.pallas_call(
        body, grid=(pl.cdiv(n, blk),),
        in_specs=[_row(blk, LAT)] * len(arrs) + [_full(a) for a in wl],
        out_specs=_row(blk, LAT),
        out_shape=jax.ShapeDtypeStruct((n, LAT), jnp.float32),
    )(*(arrs + wl))


def _rowwise(fn, out_dims, arrs, blk=256):
    n = arrs[0].shape[0]

    def body(*refs):
        ins, outs = refs[:len(arrs)], refs[len(arrs):]
        vals = fn(*[x[...] for x in ins])
        if not isinstance(vals, tuple):
            vals = (vals,)
        for o, v in zip(outs, vals):
            o[...] = v

    res = pl.pallas_call(
        body, grid=(pl.cdiv(n, blk),),
        in_specs=[_row(blk, a.shape[1]) for a in arrs],
        out_specs=[_row(blk, d) for d in out_dims],
        out_shape=[jax.ShapeDtypeStruct((n, d), jnp.float32)
                   for d in out_dims],
    )(*arrs)
    return res if len(out_dims) > 1 else res[0]


def _add2(a, b):
    return _rowwise(lambda x, y: x + y, [a.shape[1]], [a, b])


def _prep_gmp(p):
    ew, eb = p['edge']['W'], p['edge']['b']
    nw, nb = p['node']['W'], p['node']['b']
    r = lambda v: v.reshape(1, -1)
    w1 = ew[0]
    geo = w1[2 * LAT:]
    return {
        'wa': w1[:LAT], 'wb': w1[LAT:2 * LAT],
        'wg': jnp.pad(geo, ((0, LAT - geo.shape[0]), (0, 0))),
        'b1': r(eb[0]), 'w2': ew[1], 'b2': r(eb[1]), 'w3': ew[2],
        'b3': r(eb[2]), 'w4': ew[3], 'b4': r(eb[3]),
        'g': r(p['edge']['g']), 'be': r(p['edge']['be']),
        'nwh': nw[0][:LAT], 'nwag': nw[0][LAT:], 'nb1': r(nb[0]),
        'nw2': nw[1], 'nb2': r(nb[1]), 'nw3': nw[2], 'nb3': r(nb[2]),
        'nw4': nw[3], 'nb4': r(nb[3]),
        'ng': r(p['node']['g']), 'nbe': r(p['node']['be']),
    }


def kernel(h, pos, m_ids_0, m_ids_1, m_gs_0, m_gs_1, m_gs_2, params):
    f32 = jnp.float32
    h = h.astype(f32)
    ns = (h.shape[0], m_ids_0.shape[0], m_ids_1.shape[0])
    gs = (m_gs_0, m_gs_1, m_gs_2)

    lv = []
    for l in range(3):
        e = gs[l].shape[1]
        n = ns[l]
        epad = _rup(e, NW * SCK)
        npd = _rup(n, 128)
        i, j = gs[l][0], gs[l][1]
        p0 = lambda a: jnp.pad(a, (0, epad - e))
        ps = lambda a: jnp.pad(a, (0, epad - e), constant_values=n)
        lv.append(dict(E=e, Epad=epad, N=n, Np=npd, ig=p0(i), jg=p0(j),
                       i_s=ps(i), js=ps(j)))

    pos16 = jnp.pad(pos.astype(f32), ((0, 0), (0, LAT - pos.shape[1])))
    dn = [_prep_gmp(params['down'][k]) for k in range(2)]
    upp = [_prep_gmp(params['up'][k]) for k in range(2)]
    bt = _prep_gmp(params['bottom'])

    def gmp(W, hh, p16, L, skip=None):
        A, B = _matmul2(hh, W['wa'], W['wb'])
        g4 = _sc_pass(L['Epad'], 2,
                      gathers=((LAT, 0), (LAT, 1), (LAT, 0), (LAT, 1)),
                      scatters=())
        Zi, Zj, Pi, Pj = g4([L['ig'], L['jg']], [A, B, p16, p16], [])
        e = _edge_mlp(Zi, Zj, Pi, Pj, W)
        s = _sc_pass(L['Epad'], 1, gathers=(),
                     scatters=((L['Np'], LAT, 0, 'in'),))
        (aggp,) = s([L['js']], [], [e])
        n = L['N']
        return _node_mlp(hh, aggp[0][:n], aggp[1][:n], W, skip)

    w16 = jnp.ones((ns[0], LAT), f32)
    down_h, down_p, ecs = [], [], []
    hh, p16 = h, pos16
    for l in range(2):
        L = lv[l]
        n, n1 = L['N'], lv[l + 1]['N']
        h1 = gmp(dn[l], hh, p16, L)
        down_h.append(h1)
        down_p.append(p16)
        sdeg = _sc_pass(L['Epad'], 1, gathers=(),
                        scatters=((L['Np'], LAT, 0, 'o'),))
        (degp,) = sdeg([L['i_s']], [], [])
        nw16 = _rowwise(lambda w, d0, d1: w / jnp.maximum(d0 + d1, 1.0), [LAT],
                        [w16, degp[0][:n], degp[1][:n]])
        gsc = _sc_pass(L['Epad'], 2, gathers=((LAT, 0),),
                       scatters=((L['Np'], LAT, 1, ('g', 0)),))
        ws, awp = gsc([L['ig'], L['js']], [nw16], [])
        aw = _rowwise(lambda a, b: a + b + 1e-12, [LAT],
                      [awp[0][:n], awp[1][:n]])
        g3 = _sc_pass(L['Epad'], 2,
                      gathers=((LAT, 0), (LAT, 0), (LAT, 1)), scatters=())
        he, pe, awj = g3([L['ig'], L['jg']], [h1, p16, aw], [])
        ec = _rowwise(lambda a, b: a / b, [LAT], [ws, awj])
        yh, yp = _rowwise(lambda x, q, c: (x * c[:, :1], q * c[:, :1]),
                          [LAT, LAT], [he, pe, ec])
        sy = _sc_pass(L['Epad'], 1, gathers=(),
                      scatters=((L['Np'], LAT, 0, 'in'),))
        (php,) = sy([L['js']], [], [yh])
        sp = _sc_pass(L['Epad'], 1, gathers=(),
                      scatters=((L['Np'], LAT, 0, 'in'),))
        (ppp,) = sp([L['js']], [], [yp])
        hh = _add2(php[0][:n1], php[1][:n1])
        p16 = _add2(ppp[0][:n1], ppp[1][:n1])
        w16 = aw[:n1]
        ecs.append(ec)

    hh = gmp(bt, hh, p16, lv[2])

    for k in range(2):
        d = 1 - k
        L = lv[d]
        n = L['N']
        hfull = jnp.pad(hh, ((0, n - hh.shape[0]), (0, 0)))
        g1 = _sc_pass(L['Epad'], 1, gathers=((LAT, 0),), scatters=())
        (hjg,) = g1([L['jg']], [hfull], [])
        yu = _rowwise(lambda x, c: x * c[:, :1], [LAT], [hjg, ecs[d]])
        s1 = _sc_pass(L['Epad'], 1, gathers=(),
                      scatters=((L['Np'], LAT, 0, 'in'),))
        (unp,) = s1([L['i_s']], [], [yu])
        hin = _add2(unp[0][:n], unp[1][:n])
        hh = gmp(upp[k], hin, down_p[d], L, skip=down_h[d])

    return hh


# speedup vs baseline: 2.2404x; 1.8143x over previous
import functools

import jax
import jax.numpy as jnp
from jax import lax
from jax.experimental import pallas as pl
from jax.experimental.pallas import tpu as pltpu
from jax.experimental.pallas import tpu_sc as plsc

NC, NS = 2, 16
NW = NC * NS
SCK = 128
LAT = 128


def _rup(x, m):
    return (x + m - 1) // m * m


def _sc_pass(Epad, n_idx, gathers, scatters, sck=SCK):
    per_w = Epad // NW
    n_ch = per_w // sck
    n_g, n_s = len(gathers), len(scatters)
    used = sorted({g[1] for g in gathers} | {s[2] for s in scatters})
    spos = {s: k for k, s in enumerate(used)}
    in_ts = [t for t in range(n_s) if scatters[t][3] == 'in']
    vpos = {t: u for u, t in enumerate(in_ts)}
    need_ones = any(s[3] == 'o' for s in scatters)
    mesh = plsc.VectorSubcoreMesh(core_axis_name="c", subcore_axis_name="s",
                                  num_cores=NC, num_subcores=NS)

    def run(idx_arrays, tables, in_values, dep=None):
        idx_arrays = list(idx_arrays)
        if dep is not None:
            idx_arrays[0], _ = lax.optimization_barrier(
                (idx_arrays[0], dep))
        idx_r = [a.reshape(NW, n_ch, sck) for a in idx_arrays]
        zeros_l = [jnp.zeros((s[0] // NS, s[1]), jnp.float32)
                   for s in scatters]
        ones_l = [jnp.ones((sck, LAT), jnp.float32)] if need_ones else []
        emits = [k for k, g in enumerate(gathers) if g[2]]
        out_type = tuple(
            [jax.ShapeDtypeStruct((Epad, gathers[k][0]), jnp.float32)
             for k in emits]
            + [jax.ShapeDtypeStruct((NC, s[0], s[1]), jnp.float32)
               for s in scatters])
        scratch = ([pltpu.VMEM((n_ch, sck), jnp.int32) for _ in used]
                   + [pltpu.VMEM((sck, g[0]), jnp.float32)
                      for g in gathers for _ in range(2)]
                   + [pltpu.VMEM((sck, scatters[t][1]), jnp.float32)
                      for t in in_ts for _ in range(2)]
                   + ([pltpu.VMEM((sck, LAT), jnp.float32)] if need_ones else [])
                   + [pltpu.VMEM_SHARED((s[0], s[1]), jnp.float32)
                      for s in scatters]
                   + [pltpu.SemaphoreType.DMA, pltpu.SemaphoreType.DMA])

        def body(*refs):
            p = 0
            idx_in = refs[p:p + n_idx]; p += n_idx
            tbl_in = refs[p:p + n_g]; p += n_g
            val_in = refs[p:p + len(in_ts)]; p += len(in_ts)
            ones_in = refs[p:p + len(ones_l)]; p += len(ones_l)
            zer_in = refs[p:p + n_s]; p += n_s
            gout = refs[p:p + len(emits)]; p += len(emits)
            gpos = {k: u for u, k in enumerate(emits)}
            sacc = refs[p:p + n_s]; p += n_s
            idx_b = refs[p:p + len(used)]; p += len(used)
            gbuf = refs[p:p + 2 * n_g]; p += 2 * n_g
            vbuf = refs[p:p + 2 * len(in_ts)]; p += 2 * len(in_ts)
            ones_b = None
            if need_ones:
                ones_b = refs[p]; p += 1
            smem = refs[p:p + n_s]; p += n_s
            sem = refs[p:p + 2]

            cid = lax.axis_index("c")
            sid = lax.axis_index("s")
            wid = sid * NC + cid
            for t in range(n_s):
                nt = scatters[t][0] // NS
                pltpu.sync_copy(zer_in[t], smem[t].at[pl.ds(sid * nt, nt)])
            if need_ones:
                pltpu.sync_copy(ones_in[0], ones_b)
            for k in range(len(used)):
                pltpu.sync_copy(idx_in[used[k]].at[wid], idx_b[k])
            plsc.subcore_barrier()
            base = wid * per_w

            def reads(c, b):
                off = pl.multiple_of(base + c * sck, sck)
                descs = []
                for k, (D, slot, emit) in enumerate(gathers):
                    descs.append(pltpu.make_async_copy(
                        tbl_in[k].at[idx_b[spos[slot]].at[c]],
                        gbuf[2 * k + b], sem[b]))
                for u, t in enumerate(in_ts):
                    descs.append(pltpu.make_async_copy(
                        val_in[u].at[pl.ds(off, sck)], vbuf[2 * u + b],
                        sem[b]))
                return descs

            def fire(c, b):
                for d in reads(c, b):
                    d.start()

            def drain_process(c, b):
                off = pl.multiple_of(base + c * sck, sck)
                for d in reads(c, b):
                    d.wait()
                for k, (D, slot, emit) in enumerate(gathers):
                    if emit:
                        pltpu.sync_copy(gbuf[2 * k + b],
                                        gout[gpos[k]].at[pl.ds(off, sck)])
                for t, (Npt, D, slot, src) in enumerate(scatters):
                    if src == 'in':
                        sb = vbuf[2 * vpos[t] + b]
                    elif src == 'o':
                        sb = ones_b
                    else:
                        sb = gbuf[2 * src[1] + b]
                    pltpu.sync_copy(sb, smem[t].at[idx_b[spos[slot]].at[c]],
                                    add=True)

            fire(0, 0)

            def chunk(c, carry):
                nx = c + 1
                more = nx < n_ch

                @pl.when(jnp.logical_and(more, nx % 2 == 1))
                def _():
                    fire(nx, 1)

                @pl.when(jnp.logical_and(more, nx % 2 == 0))
                def _():
                    fire(nx, 0)

                @pl.when(c % 2 == 0)
                def _():
                    drain_process(c, 0)

                @pl.when(c % 2 == 1)
                def _():
                    drain_process(c, 1)

                return carry

            lax.fori_loop(0, n_ch, chunk, 0)
            plsc.subcore_barrier()
            for t in range(n_s):
                nt = scatters[t][0] // NS
                pltpu.sync_copy(smem[t].at[pl.ds(sid * nt, nt)],
                                sacc[t].at[cid, pl.ds(sid * nt, nt)])

        fn = pl.kernel(body, out_type=out_type, mesh=mesh,
                       scratch_types=scratch)
        return fn(*(idx_r + list(tables) + list(in_values) + ones_l + zeros_l))

    return run


def _full(a):
    nd = a.ndim
    return pl.BlockSpec(a.shape, lambda r: (0,) * nd)


def _row(blk, d):
    return pl.BlockSpec((blk, d), lambda r: (r, 0))


def _mlp_tail(x, w2, b2, w3, b3, w4, b4, g, be):
    x = jnp.maximum(jnp.dot(x, w2, preferred_element_type=jnp.float32) + b2, 0.)
    x = jnp.maximum(jnp.dot(x, w3, preferred_element_type=jnp.float32) + b3, 0.)
    x = jnp.dot(x, w4, preferred_element_type=jnp.float32) + b4
    mu = jnp.mean(x, axis=-1, keepdims=True)
    var = jnp.mean((x - mu) ** 2, axis=-1, keepdims=True)
    return (x - mu) * lax.rsqrt(var + 1e-5) * g + be


def _matmul2(h, wa, wb, blk=512):
    n = h.shape[0]

    def body(h_, wa_, wb_, a_, b_):
        x = h_[...]
        a_[...] = jnp.dot(x, wa_[...], preferred_element_type=jnp.float32)
        b_[...] = jnp.dot(x, wb_[...], preferred_element_type=jnp.float32)

    return pl.pallas_call(
        body, grid=(pl.cdiv(n, blk),),
        in_specs=[_row(blk, LAT), _full(wa), _full(wb)],
        out_specs=[_row(blk, LAT), _row(blk, LAT)],
        out_shape=[jax.ShapeDtypeStruct((n, LAT), jnp.float32)] * 2,
    )(h, wa, wb)


def _edge_mlp(Zi, Zj, Pi, Pj, W, blk=512):
    ep = Zi.shape[0]
    wl = [W['wg'], W['b1'], W['w2'], W['b2'], W['w3'], W['b3'], W['w4'],
          W['b4'], W['g'], W['be']]

    def body(zi, zj, pi, pj, wg, b1, w2, b2, w3, b3, w4, b4, g, be, out):
        dif = pi[...] - pj[...]
        nrm = jnp.sqrt(jnp.sum(dif * dif, axis=-1, keepdims=True))
        lane = lax.broadcasted_iota(jnp.int32, (blk, LAT), 1)
        x16 = jnp.where(lane == 3, nrm, dif)
        x = (zi[...] + zj[...] + b1[...]
             + jnp.dot(x16, wg[...], preferred_element_type=jnp.float32))
        x = jnp.maximum(x, 0.)
        out[...] = _mlp_tail(x, w2[...], b2[...], w3[...], b3[...], w4[...],
                             b4[...], g[...], be[...])

    return pl.pallas_call(
        body, grid=(ep // blk,),
        in_specs=[_row(blk, LAT)] * 4 + [_full(a) for a in wl],
        out_specs=_row(blk, LAT),
        out_shape=jax.ShapeDtypeStruct((ep, LAT), jnp.float32),
    )(Zi, Zj, Pi, Pj, *wl)


def _node_mlp(h, a0, a1, W, skip=None, blk=512):
    n = h.shape[0]
    wl = [W['nwh'], W['nwag'], W['nb1'], W['nw2'], W['nb2'], W['nw3'],
          W['nb3'], W['nw4'], W['nb4'], W['ng'], W['nbe']]
    has_skip = skip is not None

    def body(*refs):
        h_, a0_, a1_ = refs[0], refs[1], refs[2]
        k = 3
        sk = None
        if has_skip:
            sk = refs[3]
            k = 4
        wh, wag, b1, w2, b2, w3, b3, w4, b4, g, be = refs[k:k + 11]
        out = refs[k + 11]
        hv = h_[...]
        x = (jnp.dot(hv, wh[...], preferred_element_type=jnp.float32)
             + jnp.dot(a0_[...] + a1_[...], wag[...],
                       preferred_element_type=jnp.float32) + b1[...])
        x = jnp.maximum(x, 0.)
        res = _mlp_tail(x, w2[...], b2[...], w3[...], b3[...], w4[...],
                        b4[...], g[...], be[...])
        y = hv + res
        if has_skip:
            y = y + sk[...]
        out[...] = y

    arrs = [h, a0, a1] + ([skip] if has_skip else [])
    return pl.pallas_call(
        body, grid=(pl.cdiv(n, blk),),
        in_specs=[_row(blk, LAT)] * len(arrs) + [_full(a) for a in wl],
        out_specs=_row(blk, LAT),
        out_shape=jax.ShapeDtypeStruct((n, LAT), jnp.float32),
    )(*(arrs + wl))


def _rowwise(fn, out_dims, arrs, blk=256):
    n = arrs[0].shape[0]

    def body(*refs):
        ins, outs = refs[:len(arrs)], refs[len(arrs):]
        vals = fn(*[x[...] for x in ins])
        if not isinstance(vals, tuple):
            vals = (vals,)
        for o, v in zip(outs, vals):
            o[...] = v

    res = pl.pallas_call(
        body, grid=(pl.cdiv(n, blk),),
        in_specs=[_row(blk, a.shape[1]) for a in arrs],
        out_specs=[_row(blk, d) for d in out_dims],
        out_shape=[jax.ShapeDtypeStruct((n, d), jnp.float32)
                   for d in out_dims],
    )(*arrs)
    return res if len(out_dims) > 1 else res[0]


def _prep_gmp(p):
    ew, eb = p['edge']['W'], p['edge']['b']
    nw, nb = p['node']['W'], p['node']['b']
    r = lambda v: v.reshape(1, -1)
    w1 = ew[0]
    geo = w1[2 * LAT:]
    return {
        'wa': w1[:LAT], 'wb': w1[LAT:2 * LAT],
        'wg': jnp.pad(geo, ((0, LAT - geo.shape[0]), (0, 0))),
        'b1': r(eb[0]), 'w2': ew[1], 'b2': r(eb[1]), 'w3': ew[2],
        'b3': r(eb[2]), 'w4': ew[3], 'b4': r(eb[3]),
        'g': r(p['edge']['g']), 'be': r(p['edge']['be']),
        'nwh': nw[0][:LAT], 'nwag': nw[0][LAT:], 'nb1': r(nb[0]),
        'nw2': nw[1], 'nb2': r(nb[1]), 'nw3': nw[2], 'nb3': r(nb[2]),
        'nw4': nw[3], 'nb4': r(nb[3]),
        'ng': r(p['node']['g']), 'nbe': r(p['node']['be']),
    }


def kernel(h, pos, m_ids_0, m_ids_1, m_gs_0, m_gs_1, m_gs_2, params):
    f32 = jnp.float32
    h = h.astype(f32)
    ns = (h.shape[0], m_ids_0.shape[0], m_ids_1.shape[0])
    gs = (m_gs_0, m_gs_1, m_gs_2)

    lv = []
    for l in range(3):
        e = gs[l].shape[1]
        n = ns[l]
        epad = _rup(e, NW * SCK)
        npd = _rup(n, 128)
        i, j = gs[l][0], gs[l][1]
        p0 = lambda a: jnp.pad(a, (0, epad - e))
        ps = lambda a: jnp.pad(a, (0, epad - e), constant_values=n)
        lv.append(dict(E=e, Epad=epad, N=n, Np=npd, ig=p0(i), jg=p0(j),
                       i_s=ps(i), js=ps(j)))

    pos16 = jnp.pad(pos.astype(f32), ((0, 0), (0, LAT - pos.shape[1])))
    dn = [_prep_gmp(params['down'][k]) for k in range(2)]
    upp = [_prep_gmp(params['up'][k]) for k in range(2)]
    bt = _prep_gmp(params['bottom'])

    def gs_pass(L, gidx, sidx, table, dep=None):
        if L['Np'] <= 8192:
            p = _sc_pass(L['Epad'], 2, gathers=((LAT, 0, False),),
                         scatters=((L['Np'], LAT, 1, ('g', 0)),))
            (acc,) = p([gidx, sidx], [table], [], dep=dep)
        else:
            g = _sc_pass(L['Epad'], 1, gathers=((LAT, 0, True),),
                         scatters=())
            (ge,) = g([gidx], [table], [], dep=dep)
            s = _sc_pass(L['Epad'], 1, gathers=(),
                         scatters=((L['Np'], LAT, 0, 'in'),))
            (acc,) = s([sidx], [], [ge])
        return acc

    def gmp(W, hh, p16, L, skip=None):
        A, B = _matmul2(hh, W['wa'], W['wb'])
        g4 = _sc_pass(L['Epad'], 2,
                      gathers=((LAT, 0, True), (LAT, 1, True), (LAT, 0, True),
                               (LAT, 1, True)), scatters=(), sck=64)
        Zi, Zj, Pi, Pj = g4([L['ig'], L['jg']], [A, B, p16, p16], [])
        e = _edge_mlp(Zi, Zj, Pi, Pj, W)
        s = _sc_pass(L['Epad'], 1, gathers=(),
                     scatters=((L['Np'], LAT, 0, 'in'),))
        (aggp,) = s([L['js']], [], [e])
        n = L['N']
        return _node_mlp(hh, aggp[0][:n], aggp[1][:n], W, skip)

    w16 = jnp.ones((ns[0], LAT), f32)
    down_h, down_p, nws, aws = [], [], [], []
    hh, p16 = h, pos16
    for l in range(2):
        L = lv[l]
        n, n1 = L['N'], lv[l + 1]['N']
        h1 = gmp(dn[l], hh, p16, L)
        down_h.append(h1)
        down_p.append(p16)
        sdeg = _sc_pass(L['Epad'], 1, gathers=(),
                        scatters=((L['Np'], LAT, 0, 'o'),))
        (degp,) = sdeg([L['i_s']], [], [], dep=h1)
        nw16 = _rowwise(lambda w, d0, d1: w / jnp.maximum(d0 + d1, 1.0), [LAT],
                        [w16, degp[0][:n], degp[1][:n]])
        awp = gs_pass(L, L['ig'], L['js'], nw16)
        aw = _rowwise(lambda a, b: a + b + 1e-12, [LAT],
                      [awp[0][:n], awp[1][:n]])
        xh, xp = _rowwise(lambda x, q, w: (x * w[:, :1], q * w[:, :1]),
                          [LAT, LAT], [h1, p16, nw16])
        hp = gs_pass(L, L['ig'], L['js'], xh, dep=aw)
        pp = gs_pass(L, L['ig'], L['js'], xp, dep=hp)
        hh = _rowwise(lambda a, b, w: (a + b) / w, [LAT],
                      [hp[0][:n1], hp[1][:n1], aw[:n1]])
        p16 = _rowwise(lambda a, b, w: (a + b) / w, [LAT],
                       [pp[0][:n1], pp[1][:n1], aw[:n1]])
        w16 = aw[:n1]
        nws.append(nw16)
        aws.append(aw)

    hh = gmp(bt, hh, p16, lv[2])

    for k in range(2):
        d = 1 - k
        L = lv[d]
        n = L['N']
        hfull = jnp.pad(hh, ((0, n - hh.shape[0]), (0, 0)))
        hq = _rowwise(lambda x, w: x / w, [LAT], [hfull, aws[d]])
        unp = gs_pass(L, L['jg'], L['i_s'], hq)
        hin = _rowwise(lambda a, b, w: (a + b) * w[:, :1], [LAT],
                       [unp[0][:n], unp[1][:n], nws[d]])
        hh = gmp(upp[k], hin, down_p[d], L, skip=down_h[d])

    return hh


# speedup vs baseline: 2.6172x; 1.1682x over previous
import functools

import jax
import jax.numpy as jnp
from jax import lax
from jax.experimental import pallas as pl
from jax.experimental.pallas import tpu as pltpu
from jax.experimental.pallas import tpu_sc as plsc

NC, NS = 2, 16
NW = NC * NS
SCK = 128
LAT = 128


def _rup(x, m):
    return (x + m - 1) // m * m


def _sc_pass(Epad, n_idx, gathers, scatters, sck=SCK):
    per_w = Epad // NW
    n_ch = per_w // sck
    gathers = tuple(g if len(g) == 4 else g + (None,) for g in gathers)
    addpairs = [(g[3], k) for k, g in enumerate(gathers) if g[3] is not None]
    n_g, n_s = len(gathers), len(scatters)
    used = sorted({g[1] for g in gathers} | {s[2] for s in scatters})
    spos = {s: k for k, s in enumerate(used)}
    in_ts = [t for t in range(n_s) if scatters[t][3] == 'in']
    vpos = {t: u for u, t in enumerate(in_ts)}
    need_ones = any(s[3] == 'o' for s in scatters)
    mesh = plsc.VectorSubcoreMesh(core_axis_name="c", subcore_axis_name="s",
                                  num_cores=NC, num_subcores=NS)

    def run(idx_arrays, tables, in_values, dep=None):
        idx_arrays = list(idx_arrays)
        if dep is not None:
            idx_arrays[0], _ = lax.optimization_barrier(
                (idx_arrays[0], dep))
        idx_r = [a.reshape(NW, n_ch, sck) for a in idx_arrays]
        zeros_l = [jnp.zeros((s[0] // NS, s[1]), jnp.float32)
                   for s in scatters]
        ones_l = [jnp.ones((sck, LAT), jnp.float32)] if need_ones else []
        emits = [k for k, g in enumerate(gathers) if g[2]]
        out_type = tuple(
            [jax.ShapeDtypeStruct((Epad, gathers[k][0]), jnp.float32)
             for k in emits]
            + [jax.ShapeDtypeStruct((NC, s[0], s[1]), jnp.float32)
               for s in scatters])
        scratch = ([pltpu.VMEM((n_ch, sck), jnp.int32) for _ in used]
                   + [pltpu.VMEM((sck, g[0]), jnp.float32)
                      for g in gathers for _ in range(2)]
                   + [pltpu.VMEM((sck, scatters[t][1]), jnp.float32)
                      for t in in_ts for _ in range(2)]
                   + ([pltpu.VMEM((sck, LAT), jnp.float32)] if need_ones else [])
                   + [pltpu.VMEM_SHARED((s[0], s[1]), jnp.float32)
                      for s in scatters]
                   + [pltpu.SemaphoreType.DMA, pltpu.SemaphoreType.DMA])

        def body(*refs):
            p = 0
            idx_in = refs[p:p + n_idx]; p += n_idx
            tbl_in = refs[p:p + n_g]; p += n_g
            val_in = refs[p:p + len(in_ts)]; p += len(in_ts)
            ones_in = refs[p:p + len(ones_l)]; p += len(ones_l)
            zer_in = refs[p:p + n_s]; p += n_s
            gout = refs[p:p + len(emits)]; p += len(emits)
            gpos = {k: u for u, k in enumerate(emits)}
            sacc = refs[p:p + n_s]; p += n_s
            idx_b = refs[p:p + len(used)]; p += len(used)
            gbuf = refs[p:p + 2 * n_g]; p += 2 * n_g
            vbuf = refs[p:p + 2 * len(in_ts)]; p += 2 * len(in_ts)
            ones_b = None
            if need_ones:
                ones_b = refs[p]; p += 1
            smem = refs[p:p + n_s]; p += n_s
            sem = refs[p:p + 2]

            cid = lax.axis_index("c")
            sid = lax.axis_index("s")
            wid = sid * NC + cid
            for t in range(n_s):
                nt = scatters[t][0] // NS
                pltpu.sync_copy(zer_in[t], smem[t].at[pl.ds(sid * nt, nt)])
            if need_ones:
                pltpu.sync_copy(ones_in[0], ones_b)
            for k in range(len(used)):
                pltpu.sync_copy(idx_in[used[k]].at[wid], idx_b[k])
            plsc.subcore_barrier()
            base = wid * per_w

            def reads(c, b):
                off = pl.multiple_of(base + c * sck, sck)
                descs = []
                for k, (D, slot, emit, addto) in enumerate(gathers):
                    descs.append(pltpu.make_async_copy(
                        tbl_in[k].at[idx_b[spos[slot]].at[c]],
                        gbuf[2 * k + b], sem[b]))
                for u, t in enumerate(in_ts):
                    descs.append(pltpu.make_async_copy(
                        val_in[u].at[pl.ds(off, sck)], vbuf[2 * u + b],
                        sem[b]))
                return descs

            def fire(c, b):
                for d in reads(c, b):
                    d.start()

            def drain_process(c, b):
                off = pl.multiple_of(base + c * sck, sck)
                for d in reads(c, b):
                    d.wait()
                for dst, ksrc in addpairs:
                    bd, bs = gbuf[2 * dst + b], gbuf[2 * ksrc + b]

                    def addrow(r, carry, bd=bd, bs=bs, nd=gathers[dst][0]):
                        for q in range(nd // 16):
                            sl = pl.ds(q * 16, 16)
                            bd[r, sl] = bd[r, sl] + bs[r, sl]
                        return carry

                    lax.fori_loop(0, sck, addrow, 0)
                for k, (D, slot, emit, addto) in enumerate(gathers):
                    if emit:
                        pltpu.sync_copy(gbuf[2 * k + b],
                                        gout[gpos[k]].at[pl.ds(off, sck)])
                for t, (Npt, D, slot, src) in enumerate(scatters):
                    if src == 'in':
                        sb = vbuf[2 * vpos[t] + b]
                    elif src == 'o':
                        sb = ones_b
                    else:
                        sb = gbuf[2 * src[1] + b]
                    pltpu.sync_copy(sb, smem[t].at[idx_b[spos[slot]].at[c]],
                                    add=True)

            fire(0, 0)

            def chunk(c, carry):
                nx = c + 1
                more = nx < n_ch

                @pl.when(jnp.logical_and(more, nx % 2 == 1))
                def _():
                    fire(nx, 1)

                @pl.when(jnp.logical_and(more, nx % 2 == 0))
                def _():
                    fire(nx, 0)

                @pl.when(c % 2 == 0)
                def _():
                    drain_process(c, 0)

                @pl.when(c % 2 == 1)
                def _():
                    drain_process(c, 1)

                return carry

            lax.fori_loop(0, n_ch, chunk, 0)
            plsc.subcore_barrier()
            for t in range(n_s):
                nt = scatters[t][0] // NS
                pltpu.sync_copy(smem[t].at[pl.ds(sid * nt, nt)],
                                sacc[t].at[cid, pl.ds(sid * nt, nt)])

        fn = pl.kernel(body, out_type=out_type, mesh=mesh,
                       scratch_types=scratch)
        return fn(*(idx_r + list(tables) + list(in_values) + ones_l + zeros_l))

    return run


def _full(a):
    nd = a.ndim
    return pl.BlockSpec(a.shape, lambda r: (0,) * nd)


def _row(blk, d):
    return pl.BlockSpec((blk, d), lambda r: (r, 0))


def _mlp_tail(x, w2, b2, w3, b3, w4, b4, g, be):
    x = jnp.maximum(jnp.dot(x, w2, preferred_element_type=jnp.float32) + b2, 0.)
    x = jnp.maximum(jnp.dot(x, w3, preferred_element_type=jnp.float32) + b3, 0.)
    x = jnp.dot(x, w4, preferred_element_type=jnp.float32) + b4
    mu = jnp.mean(x, axis=-1, keepdims=True)
    var = jnp.mean((x - mu) ** 2, axis=-1, keepdims=True)
    return (x - mu) * lax.rsqrt(var + 1e-5) * g + be


def _matmul2(h, wa, wb, blk=512):
    n = h.shape[0]

    def body(h_, wa_, wb_, a_, b_):
        x = h_[...]
        a_[...] = jnp.dot(x, wa_[...], preferred_element_type=jnp.float32)
        b_[...] = jnp.dot(x, wb_[...], preferred_element_type=jnp.float32)

    return pl.pallas_call(
        body, grid=(pl.cdiv(n, blk),),
        in_specs=[_row(blk, LAT), _full(wa), _full(wb)],
        out_specs=[_row(blk, LAT), _row(blk, LAT)],
        out_shape=[jax.ShapeDtypeStruct((n, LAT), jnp.float32)] * 2,
    )(h, wa, wb)


def _edge_mlp(Z, Pd, W, blk=512):
    ep = Z.shape[0]
    wl = [W['wg'], W['b1'], W['w2'], W['b2'], W['w3'], W['b3'], W['w4'],
          W['b4'], W['g'], W['be']]

    def body(z, pd, wg, b1, w2, b2, w3, b3, w4, b4, g, be, out):
        dif = pd[...]
        nrm = jnp.sqrt(jnp.sum(dif * dif, axis=-1, keepdims=True))
        lane = lax.broadcasted_iota(jnp.int32, (blk, LAT), 1)
        x16 = jnp.where(lane == 3, nrm, dif)
        x = (z[...] + b1[...]
             + jnp.dot(x16, wg[...], preferred_element_type=jnp.float32))
        x = jnp.maximum(x, 0.)
        out[...] = _mlp_tail(x, w2[...], b2[...], w3[...], b3[...], w4[...],
                             b4[...], g[...], be[...])

    return pl.pallas_call(
        body, grid=(ep // blk,),
        in_specs=[_row(blk, LAT)] * 2 + [_full(a) for a in wl],
        out_specs=_row(blk, LAT),
        out_shape=jax.ShapeDtypeStruct((ep, LAT), jnp.float32),
    )(Z, Pd, *wl)


def _node_mlp(h, a0, a1, W, skip=None, blk=512):
    n = h.shape[0]
    wl = [W['nwh'], W['nwag'], W['nb1'], W['nw2'], W['nb2'], W['nw3'],
          W['nb3'], W['nw4'], W['nb4'], W['ng'], W['nbe']]
    has_skip = skip is not None

    def body(*refs):
        h_, a0_, a1_ = refs[0], refs[1], refs[2]
        k = 3
        sk = None
        if has_skip:
            sk = refs[3]
            k = 4
        wh, wag, b1, w2, b2, w3, b3, w4, b4, g, be = refs[k:k + 11]
        out = refs[k + 11]
        hv = h_[...]
        x = (jnp.dot(hv, wh[...], preferred_element_type=jnp.float32)
             + jnp.dot(a0_[...] + a1_[...], wag[...],
                       preferred_element_type=jnp.float32) + b1[...])
        x = jnp.maximum(x, 0.)
        res = _mlp_tail(x, w2[...], b2[...], w3[...], b3[...], w4[...],
                        b4[...], g[...], be[...])
        y = hv + res
        if has_skip:
            y = y + sk[...]
        out[...] = y

    arrs = [h, a0, a1] + ([skip] if has_skip else [])
    return pl.pallas_call(
        body, grid=(pl.cdiv(n, blk),),
        in_specs=[_row(blk, LAT)] * len(arrs) + [_full(a) for a in wl],
        out_specs=_row(blk, LAT),
        out_shape=jax.ShapeDtypeStruct((n, LAT), jnp.float32),
    )(*(arrs + wl))


def _rowwise(fn, out_dims, arrs, blk=256):
    n = arrs[0].shape[0]

    def body(*refs):
        ins, outs = refs[:len(arrs)], refs[len(arrs):]
        vals = fn(*[x[...] for x in ins])
        if not isinstance(vals, tuple):
            vals = (vals,)
        for o, v in zip(outs, vals):
            o[...] = v

    res = pl.pallas_call(
        body, grid=(pl.cdiv(n, blk),),
        in_specs=[_row(blk, a.shape[1]) for a in arrs],
        out_specs=[_row(blk, d) for d in out_dims],
        out_shape=[jax.ShapeDtypeStruct((n, d), jnp.float32)
                   for d in out_dims],
    )(*arrs)
    return res if len(out_dims) > 1 else res[0]


def _prep_gmp(p):
    ew, eb = p['edge']['W'], p['edge']['b']
    nw, nb = p['node']['W'], p['node']['b']
    r = lambda v: v.reshape(1, -1)
    w1 = ew[0]
    geo = w1[2 * LAT:]
    return {
        'wa': w1[:LAT], 'wb': w1[LAT:2 * LAT],
        'wg': jnp.pad(geo, ((0, LAT - geo.shape[0]), (0, 0))),
        'b1': r(eb[0]), 'w2': ew[1], 'b2': r(eb[1]), 'w3': ew[2],
        'b3': r(eb[2]), 'w4': ew[3], 'b4': r(eb[3]),
        'g': r(p['edge']['g']), 'be': r(p['edge']['be']),
        'nwh': nw[0][:LAT], 'nwag': nw[0][LAT:], 'nb1': r(nb[0]),
        'nw2': nw[1], 'nb2': r(nb[1]), 'nw3': nw[2], 'nb3': r(nb[2]),
        'nw4': nw[3], 'nb4': r(nb[3]),
        'ng': r(p['node']['g']), 'nbe': r(p['node']['be']),
    }


def kernel(h, pos, m_ids_0, m_ids_1, m_gs_0, m_gs_1, m_gs_2, params):
    f32 = jnp.float32
    h = h.astype(f32)
    ns = (h.shape[0], m_ids_0.shape[0], m_ids_1.shape[0])
    gs = (m_gs_0, m_gs_1, m_gs_2)

    lv = []
    for l in range(3):
        e = gs[l].shape[1]
        n = ns[l]
        epad = _rup(e, NW * SCK)
        npd = _rup(n, 128)
        i, j = gs[l][0], gs[l][1]
        p0 = lambda a: jnp.pad(a, (0, epad - e))
        ps = lambda a: jnp.pad(a, (0, epad - e), constant_values=n)
        lv.append(dict(E=e, Epad=epad, N=n, Np=npd, ig=p0(i), jg=p0(j),
                       i_s=ps(i), js=ps(j)))

    pos16 = jnp.pad(pos.astype(f32), ((0, 0), (0, LAT - pos.shape[1])))
    npos16 = -pos16
    dn = [_prep_gmp(params['down'][k]) for k in range(2)]
    upp = [_prep_gmp(params['up'][k]) for k in range(2)]
    bt = _prep_gmp(params['bottom'])

    def gs_pass(L, gidx, sidx, table, dep=None):
        if L['Np'] <= 8192:
            p = _sc_pass(L['Epad'], 2, gathers=((LAT, 0, False),),
                         scatters=((L['Np'], LAT, 1, ('g', 0)),))
            (acc,) = p([gidx, sidx], [table], [], dep=dep)
        else:
            g = _sc_pass(L['Epad'], 1, gathers=((LAT, 0, True),),
                         scatters=())
            (ge,) = g([gidx], [table], [], dep=dep)
            s = _sc_pass(L['Epad'], 1, gathers=(),
                         scatters=((L['Np'], LAT, 0, 'in'),))
            (acc,) = s([sidx], [], [ge])
        return acc

    def gmp(W, hh, p16, np16, L, skip=None):
        A, B = _matmul2(hh, W['wa'], W['wb'])
        g4 = _sc_pass(L['Epad'], 2,
                      gathers=((LAT, 0, True), (LAT, 1, False, 0),
                               (LAT, 0, True), (LAT, 1, False, 2)),
                      scatters=(), sck=64)
        Z, Pd = g4([L['ig'], L['jg']], [A, B, p16, np16], [])
        e = _edge_mlp(Z, Pd, W)
        s = _sc_pass(L['Epad'], 1, gathers=(),
                     scatters=((L['Np'], LAT, 0, 'in'),))
        (aggp,) = s([L['js']], [], [e])
        n = L['N']
        return _node_mlp(hh, aggp[0][:n], aggp[1][:n], W, skip)

    w16 = jnp.ones((ns[0], LAT), f32)
    down_h, down_p, down_np, nws, aws = [], [], [], [], []
    hh, p16, np16 = h, pos16, npos16
    for l in range(2):
        L = lv[l]
        n, n1 = L['N'], lv[l + 1]['N']
        h1 = gmp(dn[l], hh, p16, np16, L)
        down_h.append(h1)
        down_p.append(p16)
        down_np.append(np16)
        sdeg = _sc_pass(L['Epad'], 1, gathers=(),
                        scatters=((L['Np'], LAT, 0, 'o'),))
        (degp,) = sdeg([L['i_s']], [], [], dep=h1)
        nw16 = _rowwise(lambda w, d0, d1: w / jnp.maximum(d0 + d1, 1.0), [LAT],
                        [w16, degp[0][:n], degp[1][:n]])
        awp = gs_pass(L, L['ig'], L['js'], nw16)
        aw = _rowwise(lambda a, b: a + b + 1e-12, [LAT],
                      [awp[0][:n], awp[1][:n]])
        xh, xp = _rowwise(lambda x, q, w: (x * w[:, :1], q * w[:, :1]),
                          [LAT, LAT], [h1, p16, nw16])
        hp = gs_pass(L, L['ig'], L['js'], xh, dep=aw)
        pp = gs_pass(L, L['ig'], L['js'], xp, dep=hp)
        hh = _rowwise(lambda a, b, w: (a + b) / w, [LAT],
                      [hp[0][:n1], hp[1][:n1], aw[:n1]])
        p16, np16 = _rowwise(lambda a, b, w: ((a + b) / w, -((a + b) / w)),
                             [LAT, LAT], [pp[0][:n1], pp[1][:n1], aw[:n1]])
        w16 = aw[:n1]
        nws.append(nw16)
        aws.append(aw)

    hh = gmp(bt, hh, p16, np16, lv[2])

    for k in range(2):
        d = 1 - k
        L = lv[d]
        n = L['N']
        hfull = jnp.pad(hh, ((0, n - hh.shape[0]), (0, 0)))
        hq = _rowwise(lambda x, w: x / w, [LAT], [hfull, aws[d]])
        unp = gs_pass(L, L['jg'], L['i_s'], hq)
        hin = _rowwise(lambda a, b, w: (a + b) * w[:, :1], [LAT],
                       [unp[0][:n], unp[1][:n], nws[d]])
        hh = gmp(upp[k], hin, down_p[d], down_np[d], L, skip=down_h[d])

    return hh
